# Initial kernel scaffold; baseline (speedup 1.0000x reference)
import functools

import jax
import jax.numpy as jnp
from jax import lax
from jax.experimental import pallas as pl
from jax.experimental.pallas import tpu as pltpu
from jax.experimental.pallas import tpu_sc as plsc

N = 10000
NP = 10240
E = 320000
D = 128
H = 64
C = 10
CP = 16

NC = 2
NS = 16
NW = NC * NS
EPW = E // NW
B = 80
NCH = EPW // B
NPS = NP // NS

_mesh = plsc.VectorSubcoreMesh(core_axis_name="c", subcore_axis_name="s")
_f32 = jnp.float32


def _deg_body(dst_hbm, w_hbm, out_hbm, dst_v, w_v, z_v, acc_sh):
    c = lax.axis_index("c")
    s = lax.axis_index("s")
    wid = s * NC + c

    @pl.loop(0, NPS, step=16)
    def _(i):
        z_v[pl.ds(i, 16)] = jnp.zeros((16,), _f32)

    pltpu.sync_copy(z_v, acc_sh.at[pl.ds(s * NPS, NPS)])
    plsc.subcore_barrier()

    pltpu.sync_copy(dst_hbm.at[wid], dst_v)
    pltpu.sync_copy(w_hbm.at[wid], w_v)

    @pl.loop(0, NCH)
    def _(ci):
        pltpu.sync_copy(w_v.at[ci], acc_sh.at[dst_v.at[ci]], add=True)

    plsc.subcore_barrier()
    pltpu.sync_copy(acc_sh.at[pl.ds(s * NPS, NPS)],
                    out_hbm.at[c, pl.ds(s * NPS, NPS)])


@functools.partial(
    pl.kernel,
    out_type=jax.ShapeDtypeStruct((NC, NP), _f32),
    mesh=_mesh,
    scratch_types=[
        pltpu.VMEM((NCH, B), jnp.int32),
        pltpu.VMEM((NCH, B), _f32),
        pltpu.VMEM((NPS,), _f32),
        pltpu.VMEM_SHARED((NP,), _f32),
    ],
)
def _deg_kernel(dst_hbm, w_hbm, out_hbm, dst_v, w_v, z_v, acc_sh):
    _deg_body(dst_hbm, w_hbm, out_hbm, dst_v, w_v, z_v, acc_sh)


def _msg_body(wd, hs_hbm, src_hbm, dst_hbm, w_hbm, out_hbm,
              src_v, dst_v, w_v, rows, z_v, acc_sh):
    c = lax.axis_index("c")
    s = lax.axis_index("s")
    wid = s * NC + c

    @pl.loop(0, B)
    def _(r):
        for q in range(wd // 16):
            z_v[r, pl.ds(q * 16, 16)] = jnp.zeros((16,), _f32)

    @pl.loop(0, NPS // B)
    def _(j):
        pltpu.sync_copy(z_v, acc_sh.at[pl.ds(s * NPS + j * B, B)])

    plsc.subcore_barrier()

    pltpu.sync_copy(src_hbm.at[wid], src_v)
    pltpu.sync_copy(dst_hbm.at[wid], dst_v)
    pltpu.sync_copy(w_hbm.at[wid], w_v)

    @pl.loop(0, NCH)
    def _(ci):
        pltpu.sync_copy(hs_hbm.at[src_v.at[ci]], rows)
        ci_idx = jnp.full((16,), ci, jnp.int32)
        for r in range(B):
            splat = plsc.load_gather(
                w_v, [ci_idx, jnp.full((16,), r, jnp.int32)])
            for q in range(wd // 16):
                rows[r, pl.ds(q * 16, 16)] = rows[r, pl.ds(q * 16, 16)] * splat
        pltpu.sync_copy(rows, acc_sh.at[dst_v.at[ci]], add=True)

    plsc.subcore_barrier()
    pltpu.sync_copy(acc_sh.at[pl.ds(s * NPS, NPS)],
                    out_hbm.at[c, pl.ds(s * NPS, NPS)])


def _make_msg_kernel(wd):
    @functools.partial(
        pl.kernel,
        out_type=jax.ShapeDtypeStruct((NC, NP, wd), _f32),
        mesh=_mesh,
        scratch_types=[
            pltpu.VMEM((NCH, B), jnp.int32),
            pltpu.VMEM((NCH, B), jnp.int32),
            pltpu.VMEM((NCH, B), _f32),
            pltpu.VMEM((B, wd), _f32),
            pltpu.VMEM((B, wd), _f32),
            pltpu.VMEM_SHARED((NP, wd), _f32),
        ],
    )
    def _k(hs_hbm, src_hbm, dst_hbm, w_hbm, out_hbm,
           src_v, dst_v, w_v, rows, z_v, acc_sh):
        _msg_body(wd, hs_hbm, src_hbm, dst_hbm, w_hbm, out_hbm,
                  src_v, dst_v, w_v, rows, z_v, acc_sh)
    return _k


_msg_kernel_h = _make_msg_kernel(H)
_msg_kernel_c = _make_msg_kernel(CP)


def _tc1_body(x_ref, w1_ref, degp_ref, hs_ref):
    deg = degp_ref[0] + degp_ref[1] + 1.0
    dinv = lax.rsqrt(deg)
    h = jnp.dot(x_ref[...], w1_ref[...], preferred_element_type=_f32)
    hs_ref[...] = h * dinv


def _tc2_body(acc_ref, hs_ref, degp_ref, w2_ref, b1_ref, hs2_ref):
    deg = degp_ref[0] + degp_ref[1] + 1.0
    dinv = lax.rsqrt(deg)
    t = jax.nn.relu(dinv * (acc_ref[0] + acc_ref[1] + hs_ref[...])
                    + b1_ref[...])
    hs2_ref[...] = jnp.dot(t, w2_ref[...], preferred_element_type=_f32) * dinv


def _tc3_body(acc_ref, hs2_ref, degp_ref, b2_ref, lp_ref, xo_ref):
    deg = degp_ref[0] + degp_ref[1] + 1.0
    dinv = lax.rsqrt(deg)
    xo = dinv * (acc_ref[0] + acc_ref[1] + hs2_ref[...]) + b2_ref[...]
    col = lax.broadcasted_iota(jnp.int32, (NP, CP), 1)
    masked = jnp.where(col < C, xo, -1e30)
    m = jnp.max(masked, axis=1, keepdims=True)
    ssum = jnp.sum(jnp.exp(masked - m), axis=1, keepdims=True)
    lp_ref[...] = xo - m - jnp.log(ssum)
    xo_ref[...] = xo


def kernel(x, edge_index, e_w, idx, W1, b1, W2, b2):
    w = jnp.where(idx == 0, jnp.ones((E,), x.dtype), e_w)
    src3 = edge_index[0].reshape(NW, NCH, B)
    dst3 = edge_index[1].reshape(NW, NCH, B)
    w3 = w.reshape(NW, NCH, B)

    x_pad = jnp.pad(x, ((0, NP - N), (0, 0)))
    w2p = jnp.pad(W2, ((0, 0), (0, CP - C)))
    b1r = b1.reshape(1, H)
    b2r = jnp.pad(b2, (0, CP - C)).reshape(1, CP)

    degp = _deg_kernel(dst3, w3)
    degp3 = degp.reshape(NC, NP, 1)

    hs = pl.pallas_call(
        _tc1_body,
        out_shape=jax.ShapeDtypeStruct((NP, H), _f32),
    )(x_pad, W1, degp3)

    acc1 = _msg_kernel_h(hs, src3, dst3, w3)

    hs2 = pl.pallas_call(
        _tc2_body,
        out_shape=jax.ShapeDtypeStruct((NP, CP), _f32),
    )(acc1, hs, degp3, w2p, b1r)

    acc2 = _msg_kernel_c(hs2, src3, dst3, w3)

    lp, xo = pl.pallas_call(
        _tc3_body,
        out_shape=[jax.ShapeDtypeStruct((NP, CP), _f32),
                   jax.ShapeDtypeStruct((NP, CP), _f32)],
    )(acc2, hs2, degp3, b2r)

    log_probs = lp[:N, :C]
    x_out = xo[:N, :C]
    preg = jnp.asarray(0.0, dtype=_f32)
    return (log_probs, x_out, preg)


# speedup vs baseline: 19.7322x; 19.7322x over previous
import dataclasses
import functools

import jax
import jax.numpy as jnp
from jax import lax
from jax.experimental import pallas as pl
from jax.experimental.pallas import tpu as pltpu
from jax.experimental.pallas import tpu_sc as plsc

N = 10000
NP = 10240
E = 320000
D = 128
H = 64
C = 10
CP = 16

NC = 2
NS = 16
NW = NC * NS
EPW = E // NW
B = 80
NCH = EPW // B
NPS = NP // NS

_mesh = plsc.VectorSubcoreMesh(core_axis_name="c", subcore_axis_name="s")
_f32 = jnp.float32

_sc_params = pltpu.CompilerParams(
    needs_layout_passes=False, use_tc_tiling_on_sc=False)


def _deg_body(dst_hbm, w_hbm, out_hbm, dst_v, w_v, z_v, acc_sh):
    c = lax.axis_index("c")
    s = lax.axis_index("s")
    wid = s * NC + c

    @pl.loop(0, NPS, step=16)
    def _(i):
        z_v[pl.ds(i, 16)] = jnp.zeros((16,), _f32)

    pltpu.sync_copy(z_v, acc_sh.at[pl.ds(s * NPS, NPS)])
    plsc.subcore_barrier()

    pltpu.sync_copy(dst_hbm.at[wid], dst_v)
    pltpu.sync_copy(w_hbm.at[wid], w_v)

    @pl.loop(0, NCH)
    def _(ci):
        pltpu.sync_copy(w_v.at[ci], acc_sh.at[dst_v.at[ci]], add=True)

    plsc.subcore_barrier()
    pltpu.sync_copy(acc_sh.at[pl.ds(s * NPS, NPS)],
                    out_hbm.at[c, pl.ds(s * NPS, NPS)])


@functools.partial(
    pl.kernel,
    out_type=jax.ShapeDtypeStruct((NC, NP), _f32),
    mesh=_mesh,
    scratch_types=[
        pltpu.VMEM((NCH, B), jnp.int32),
        pltpu.VMEM((NCH, B), _f32),
        pltpu.VMEM((NPS,), _f32),
        pltpu.VMEM_SHARED((NP,), _f32),
    ],
    compiler_params=_sc_params,
)
def _deg_kernel(dst_hbm, w_hbm, out_hbm, dst_v, w_v, z_v, acc_sh):
    _deg_body(dst_hbm, w_hbm, out_hbm, dst_v, w_v, z_v, acc_sh)


def _msg_body(wd, hs_hbm, src_hbm, dst_hbm, w_hbm, out_hbm,
              src_v, dst_v, w_v, rows, z_v, acc_sh):
    c = lax.axis_index("c")
    s = lax.axis_index("s")
    wid = s * NC + c

    @pl.loop(0, B)
    def _(r):
        for q in range(wd // 16):
            z_v[r, pl.ds(q * 16, 16)] = jnp.zeros((16,), _f32)

    @pl.loop(0, NPS // B)
    def _(j):
        pltpu.sync_copy(z_v, acc_sh.at[pl.ds(s * NPS + j * B, B)])

    plsc.subcore_barrier()

    pltpu.sync_copy(src_hbm.at[wid], src_v)
    pltpu.sync_copy(dst_hbm.at[wid], dst_v)
    pltpu.sync_copy(w_hbm.at[wid], w_v)

    @pl.loop(0, NCH)
    def _(ci):
        pltpu.sync_copy(hs_hbm.at[src_v.at[ci]], rows)
        ci_idx = jnp.full((16,), ci, jnp.int32)
        for r in range(B):
            splat = plsc.load_gather(
                w_v, [ci_idx, jnp.full((16,), r, jnp.int32)])
            for q in range(wd // 16):
                rows[r, pl.ds(q * 16, 16)] = rows[r, pl.ds(q * 16, 16)] * splat
        pltpu.sync_copy(rows, acc_sh.at[dst_v.at[ci]], add=True)

    plsc.subcore_barrier()
    pltpu.sync_copy(acc_sh.at[pl.ds(s * NPS, NPS)],
                    out_hbm.at[c, pl.ds(s * NPS, NPS)])


def _make_msg_kernel(wd):
    @functools.partial(
        pl.kernel,
        out_type=jax.ShapeDtypeStruct((NC, NP, wd), _f32),
        mesh=_mesh,
        scratch_types=[
            pltpu.VMEM((NCH, B), jnp.int32),
            pltpu.VMEM((NCH, B), jnp.int32),
            pltpu.VMEM((NCH, B), _f32),
            pltpu.VMEM((B, wd), _f32),
            pltpu.VMEM((B, wd), _f32),
            pltpu.VMEM_SHARED((NP, wd), _f32),
        ],
        compiler_params=_sc_params,
    )
    def _k(hs_hbm, src_hbm, dst_hbm, w_hbm, out_hbm,
           src_v, dst_v, w_v, rows, z_v, acc_sh):
        _msg_body(wd, hs_hbm, src_hbm, dst_hbm, w_hbm, out_hbm,
                  src_v, dst_v, w_v, rows, z_v, acc_sh)
    return _k


_msg_kernel_h = _make_msg_kernel(H)
_msg_kernel_c = _make_msg_kernel(CP)


def _tc1_body(x_ref, w1_ref, degp_ref, hs_ref):
    deg = degp_ref[0] + degp_ref[1] + 1.0
    dinv = lax.rsqrt(deg)
    h = jnp.dot(x_ref[...], w1_ref[...], preferred_element_type=_f32)
    hs_ref[...] = h * dinv


def _tc2_body(acc_ref, hs_ref, degp_ref, w2_ref, b1_ref, hs2_ref):
    deg = degp_ref[0] + degp_ref[1] + 1.0
    dinv = lax.rsqrt(deg)
    t = jax.nn.relu(dinv * (acc_ref[0] + acc_ref[1] + hs_ref[...])
                    + b1_ref[...])
    hs2_ref[...] = jnp.dot(t, w2_ref[...], preferred_element_type=_f32) * dinv


def _tc3_body(acc_ref, hs2_ref, degp_ref, b2_ref, lp_ref, xo_ref):
    deg = degp_ref[0] + degp_ref[1] + 1.0
    dinv = lax.rsqrt(deg)
    xo = dinv * (acc_ref[0] + acc_ref[1] + hs2_ref[...]) + b2_ref[...]
    col = lax.broadcasted_iota(jnp.int32, (NP, CP), 1)
    masked = jnp.where(col < C, xo, -1e30)
    m = jnp.max(masked, axis=1, keepdims=True)
    ssum = jnp.sum(jnp.exp(masked - m), axis=1, keepdims=True)
    lp_ref[...] = xo - m - jnp.log(ssum)
    xo_ref[...] = xo


def kernel(x, edge_index, e_w, idx, W1, b1, W2, b2):
    w = jnp.where(idx == 0, jnp.ones((E,), x.dtype), e_w)
    src3 = edge_index[0].reshape(NW, NCH, B)
    dst3 = edge_index[1].reshape(NW, NCH, B)
    w3 = w.reshape(NW, NCH, B)

    x_pad = jnp.pad(x, ((0, NP - N), (0, 0)))
    w2p = jnp.pad(W2, ((0, 0), (0, CP - C)))
    b1r = b1.reshape(1, H)
    b2r = jnp.pad(b2, (0, CP - C)).reshape(1, CP)

    degp = _deg_kernel(dst3, w3)
    degp3 = degp.reshape(NC, NP, 1)

    hs = pl.pallas_call(
        _tc1_body,
        out_shape=jax.ShapeDtypeStruct((NP, H), _f32),
    )(x_pad, W1, degp3)

    acc1 = _msg_kernel_h(hs, src3, dst3, w3)

    hs2 = pl.pallas_call(
        _tc2_body,
        out_shape=jax.ShapeDtypeStruct((NP, CP), _f32),
    )(acc1, hs, degp3, w2p, b1r)

    acc2 = _msg_kernel_c(hs2, src3, dst3, w3)

    lp, xo = pl.pallas_call(
        _tc3_body,
        out_shape=[jax.ShapeDtypeStruct((NP, CP), _f32),
                   jax.ShapeDtypeStruct((NP, CP), _f32)],
    )(acc2, hs2, degp3, b2r)

    log_probs = lp[:N, :C]
    x_out = xo[:N, :C]
    preg = jnp.asarray(0.0, dtype=_f32)
    return (log_probs, x_out, preg)


# speedup vs baseline: 20.7611x; 1.0521x over previous
import dataclasses
import functools

import jax
import jax.numpy as jnp
from jax import lax
from jax.experimental import pallas as pl
from jax.experimental.pallas import tpu as pltpu
from jax.experimental.pallas import tpu_sc as plsc

N = 10000
NP = 10240
E = 320000
D = 128
H = 64
C = 10
CP = 16

NC = 2
NS = 16
NW = NC * NS
EPW = E // NW
B = 80
NCH = EPW // B
NPS = NP // NS

_mesh = plsc.VectorSubcoreMesh(core_axis_name="c", subcore_axis_name="s")
_f32 = jnp.float32

_sc_params = pltpu.CompilerParams(
    needs_layout_passes=False, use_tc_tiling_on_sc=False)


def _deg_body(dst_hbm, w_hbm, out_hbm, dst_v, w_v, z_v, acc_sh):
    c = lax.axis_index("c")
    s = lax.axis_index("s")
    wid = s * NC + c

    @pl.loop(0, NPS, step=16)
    def _(i):
        z_v[pl.ds(i, 16)] = jnp.zeros((16,), _f32)

    pltpu.sync_copy(z_v, acc_sh.at[pl.ds(s * NPS, NPS)])
    plsc.subcore_barrier()

    pltpu.sync_copy(dst_hbm.at[wid], dst_v)
    pltpu.sync_copy(w_hbm.at[wid], w_v)

    @pl.loop(0, NCH)
    def _(ci):
        pltpu.sync_copy(w_v.at[ci], acc_sh.at[dst_v.at[ci]], add=True)

    plsc.subcore_barrier()
    pltpu.sync_copy(acc_sh.at[pl.ds(s * NPS, NPS)],
                    out_hbm.at[c, pl.ds(s * NPS, NPS)])


@functools.partial(
    pl.kernel,
    out_type=jax.ShapeDtypeStruct((NC, NP), _f32),
    mesh=_mesh,
    scratch_types=[
        pltpu.VMEM((NCH, B), jnp.int32),
        pltpu.VMEM((NCH, B), _f32),
        pltpu.VMEM((NPS,), _f32),
        pltpu.VMEM_SHARED((NP,), _f32),
    ],
    compiler_params=_sc_params,
)
def _deg_kernel(dst_hbm, w_hbm, out_hbm, dst_v, w_v, z_v, acc_sh):
    _deg_body(dst_hbm, w_hbm, out_hbm, dst_v, w_v, z_v, acc_sh)


NBUF = 5


def _msg_body(wd, hs_hbm, src_hbm, dst_hbm, w_hbm, out_hbm,
              src_v, dst_v, w_v, rows, z_v, acc_sh, gsem, ssem):
    c = lax.axis_index("c")
    s = lax.axis_index("s")
    wid = s * NC + c

    @pl.loop(0, B)
    def _(r):
        for q in range(wd // 16):
            z_v[r, pl.ds(q * 16, 16)] = jnp.zeros((16,), _f32)

    @pl.loop(0, NPS // B)
    def _(j):
        pltpu.sync_copy(z_v, acc_sh.at[pl.ds(s * NPS + j * B, B)])

    plsc.subcore_barrier()

    pltpu.sync_copy(src_hbm.at[wid], src_v)
    pltpu.sync_copy(dst_hbm.at[wid], dst_v)
    pltpu.sync_copy(w_hbm.at[wid], w_v)

    def gstart(ci, b):
        pltpu.async_copy(hs_hbm.at[src_v.at[ci]], rows.at[b], gsem.at[b])

    def gwait(b):
        pltpu.make_async_copy(hs_hbm.at[src_v.at[0]], rows.at[b],
                              gsem.at[b]).wait()

    def sstart(ci, b):
        pltpu.async_copy(rows.at[b], acc_sh.at[dst_v.at[ci]], ssem.at[b],
                         add=True)

    def swait(b):
        pltpu.make_async_copy(rows.at[b], acc_sh.at[dst_v.at[0]],
                              ssem.at[b]).wait()

    def scale_ci(ci, b):
        for r in range(B):
            splat = plsc.load_gather(
                w_v, [jnp.full((16,), ci, jnp.int32),
                      jnp.full((16,), r, jnp.int32)])
            for q in range(wd // 16):
                rows[b, r, pl.ds(q * 16, 16)] = (
                    rows[b, r, pl.ds(q * 16, 16)] * splat)

    for b in range(NBUF):
        gstart(b, b)

    @pl.loop(0, NCH - NBUF, step=NBUF)
    def _(c0):
        for b in range(NBUF):
            gwait(b)
            scale_ci(c0 + b, b)
            sstart(c0 + b, b)
        for b in range(NBUF):
            swait(b)
            gstart(c0 + NBUF + b, b)

    for b in range(NBUF):
        gwait(b)
        scale_ci(NCH - NBUF + b, b)
        sstart(NCH - NBUF + b, b)
    for b in range(NBUF):
        swait(b)

    plsc.subcore_barrier()
    pltpu.sync_copy(acc_sh.at[pl.ds(s * NPS, NPS)],
                    out_hbm.at[c, pl.ds(s * NPS, NPS)])


def _make_msg_kernel(wd):
    @functools.partial(
        pl.kernel,
        out_type=jax.ShapeDtypeStruct((NC, NP, wd), _f32),
        mesh=_mesh,
        scratch_types=[
            pltpu.VMEM((NCH, B), jnp.int32),
            pltpu.VMEM((NCH, B), jnp.int32),
            pltpu.VMEM((NCH, B), _f32),
            pltpu.VMEM((NBUF, B, wd), _f32),
            pltpu.VMEM((B, wd), _f32),
            pltpu.VMEM_SHARED((NP, wd), _f32),
            pltpu.SemaphoreType.DMA((NBUF,)),
            pltpu.SemaphoreType.DMA((NBUF,)),
        ],
        compiler_params=_sc_params,
    )
    def _k(hs_hbm, src_hbm, dst_hbm, w_hbm, out_hbm,
           src_v, dst_v, w_v, rows, z_v, acc_sh, gsem, ssem):
        _msg_body(wd, hs_hbm, src_hbm, dst_hbm, w_hbm, out_hbm,
                  src_v, dst_v, w_v, rows, z_v, acc_sh, gsem, ssem)
    return _k


_msg_kernel_h = _make_msg_kernel(H)
_msg_kernel_c = _make_msg_kernel(CP)


def _tc1_body(x_ref, w1_ref, degp_ref, hs_ref):
    deg = degp_ref[0] + degp_ref[1] + 1.0
    dinv = lax.rsqrt(deg)
    h = jnp.dot(x_ref[...], w1_ref[...], preferred_element_type=_f32)
    hs_ref[...] = h * dinv


def _tc2_body(acc_ref, hs_ref, degp_ref, w2_ref, b1_ref, hs2_ref):
    deg = degp_ref[0] + degp_ref[1] + 1.0
    dinv = lax.rsqrt(deg)
    t = jax.nn.relu(dinv * (acc_ref[0] + acc_ref[1] + hs_ref[...])
                    + b1_ref[...])
    hs2_ref[...] = jnp.dot(t, w2_ref[...], preferred_element_type=_f32) * dinv


def _tc3_body(acc_ref, hs2_ref, degp_ref, b2_ref, lp_ref, xo_ref):
    deg = degp_ref[0] + degp_ref[1] + 1.0
    dinv = lax.rsqrt(deg)
    xo = dinv * (acc_ref[0] + acc_ref[1] + hs2_ref[...]) + b2_ref[...]
    col = lax.broadcasted_iota(jnp.int32, (NP, CP), 1)
    masked = jnp.where(col < C, xo, -1e30)
    m = jnp.max(masked, axis=1, keepdims=True)
    ssum = jnp.sum(jnp.exp(masked - m), axis=1, keepdims=True)
    lp_ref[...] = xo - m - jnp.log(ssum)
    xo_ref[...] = xo


def kernel(x, edge_index, e_w, idx, W1, b1, W2, b2):
    w = jnp.where(idx == 0, jnp.ones((E,), x.dtype), e_w)
    src3 = edge_index[0].reshape(NW, NCH, B)
    dst3 = edge_index[1].reshape(NW, NCH, B)
    w3 = w.reshape(NW, NCH, B)

    x_pad = jnp.pad(x, ((0, NP - N), (0, 0)))
    w2p = jnp.pad(W2, ((0, 0), (0, CP - C)))
    b1r = b1.reshape(1, H)
    b2r = jnp.pad(b2, (0, CP - C)).reshape(1, CP)

    degp = _deg_kernel(dst3, w3)
    degp3 = degp.reshape(NC, NP, 1)

    hs = pl.pallas_call(
        _tc1_body,
        out_shape=jax.ShapeDtypeStruct((NP, H), _f32),
    )(x_pad, W1, degp3)

    acc1 = _msg_kernel_h(hs, src3, dst3, w3)

    hs2 = pl.pallas_call(
        _tc2_body,
        out_shape=jax.ShapeDtypeStruct((NP, CP), _f32),
    )(acc1, hs, degp3, w2p, b1r)

    acc2 = _msg_kernel_c(hs2, src3, dst3, w3)

    lp, xo = pl.pallas_call(
        _tc3_body,
        out_shape=[jax.ShapeDtypeStruct((NP, CP), _f32),
                   jax.ShapeDtypeStruct((NP, CP), _f32)],
    )(acc2, hs2, degp3, b2r)

    log_probs = lp[:N, :C]
    x_out = xo[:N, :C]
    preg = jnp.asarray(0.0, dtype=_f32)
    return (log_probs, x_out, preg)


# speedup vs baseline: 21.2863x; 1.0253x over previous
import dataclasses
import functools

import jax
import jax.numpy as jnp
from jax import lax
from jax.experimental import pallas as pl
from jax.experimental.pallas import tpu as pltpu
from jax.experimental.pallas import tpu_sc as plsc

N = 10000
NP = 10240
E = 320000
D = 128
H = 64
C = 10
CP = 16

NC = 2
NS = 16
NW = NC * NS
EPW = E // NW
B = 80
NCH = EPW // B
NPS = NP // NS

_mesh = plsc.VectorSubcoreMesh(core_axis_name="c", subcore_axis_name="s")
_f32 = jnp.float32

_sc_params = pltpu.CompilerParams(
    needs_layout_passes=False, use_tc_tiling_on_sc=False)


def _unpack_dst(sd_v, dst_v):
    @pl.loop(0, NCH)
    def _(ci):
        for g in range(B // 16):
            v = sd_v[ci, pl.ds(g * 16, 16)]
            dst_v[ci, pl.ds(g * 16, 16)] = lax.shift_right_logical(v, 16)


def _unpack_src_dst(sd_v, src_v, dst_v):
    @pl.loop(0, NCH)
    def _(ci):
        for g in range(B // 16):
            v = sd_v[ci, pl.ds(g * 16, 16)]
            src_v[ci, pl.ds(g * 16, 16)] = v & 0xFFFF
            dst_v[ci, pl.ds(g * 16, 16)] = lax.shift_right_logical(v, 16)


def _deg_body(sd_hbm, w_hbm, out_hbm, sd_v, dst_v, w_v, z_v, acc_sh):
    c = lax.axis_index("c")
    s = lax.axis_index("s")
    wid = s * NC + c

    @pl.loop(0, NPS, step=16)
    def _(i):
        z_v[pl.ds(i, 16)] = jnp.zeros((16,), _f32)

    pltpu.sync_copy(z_v, acc_sh.at[pl.ds(s * NPS, NPS)])
    plsc.subcore_barrier()

    pltpu.sync_copy(sd_hbm.at[wid], sd_v)
    _unpack_dst(sd_v, dst_v)
    pltpu.sync_copy(w_hbm.at[wid], w_v)

    @pl.loop(0, NCH)
    def _(ci):
        pltpu.sync_copy(w_v.at[ci], acc_sh.at[dst_v.at[ci]], add=True)

    plsc.subcore_barrier()
    pltpu.sync_copy(acc_sh.at[pl.ds(s * NPS, NPS)],
                    out_hbm.at[c, pl.ds(s * NPS, NPS)])


@functools.partial(
    pl.kernel,
    out_type=jax.ShapeDtypeStruct((NC, NP), _f32),
    mesh=_mesh,
    scratch_types=[
        pltpu.VMEM((NCH, B), jnp.int32),
        pltpu.VMEM((NCH, B), jnp.int32),
        pltpu.VMEM((NCH, B), _f32),
        pltpu.VMEM((NPS,), _f32),
        pltpu.VMEM_SHARED((NP,), _f32),
    ],
    compiler_params=_sc_params,
)
def _deg_kernel(sd_hbm, w_hbm, out_hbm, sd_v, dst_v, w_v, z_v, acc_sh):
    _deg_body(sd_hbm, w_hbm, out_hbm, sd_v, dst_v, w_v, z_v, acc_sh)


NBUF = 5


def _msg_body(wd, hs_hbm, sd_hbm, w_hbm, out_hbm,
              sd_v, src_v, dst_v, w_v, rows, z_v, acc_sh, hs_sh,
              isem, gsem, ssem):
    c = lax.axis_index("c")
    s = lax.axis_index("s")
    wid = s * NC + c

    pltpu.sync_copy(hs_hbm.at[pl.ds(s * NPS, NPS)],
                    hs_sh.at[pl.ds(s * NPS, NPS)])

    @pl.loop(0, B)
    def _(r):
        for q in range(wd // 16):
            z_v[r, pl.ds(q * 16, 16)] = jnp.zeros((16,), _f32)

    @pl.loop(0, NPS // B)
    def _(j):
        pltpu.sync_copy(z_v, acc_sh.at[pl.ds(s * NPS + j * B, B)])

    plsc.subcore_barrier()

    def istart(ci, b):
        pltpu.async_copy(sd_hbm.at[wid, ci], sd_v.at[b], isem.at[b])
        pltpu.async_copy(w_hbm.at[wid, ci], w_v.at[b], isem.at[b])

    def iwait(b):
        pltpu.make_async_copy(sd_hbm.at[0, 0], sd_v.at[b], isem.at[b]).wait()
        pltpu.make_async_copy(w_hbm.at[0, 0], w_v.at[b], isem.at[b]).wait()

    def decode(b):
        for g in range(B // 16):
            v = sd_v[b, pl.ds(g * 16, 16)]
            src_v[b, pl.ds(g * 16, 16)] = v & 0xFFFF
            dst_v[b, pl.ds(g * 16, 16)] = lax.shift_right_logical(v, 16)

    def gstart(b):
        pltpu.async_copy(hs_sh.at[src_v.at[b]], rows.at[b], gsem.at[b])

    def gwait(b):
        pltpu.make_async_copy(hs_sh.at[src_v.at[0]], rows.at[b],
                              gsem.at[b]).wait()

    def sstart(b):
        pltpu.async_copy(rows.at[b], acc_sh.at[dst_v.at[b]], ssem.at[b],
                         add=True)

    def swait(b):
        pltpu.make_async_copy(rows.at[b], acc_sh.at[dst_v.at[0]],
                              ssem.at[b]).wait()

    def scale(b):
        for r in range(B):
            splat = plsc.load_gather(
                w_v, [jnp.full((16,), b, jnp.int32),
                      jnp.full((16,), r, jnp.int32)])
            for q in range(wd // 16):
                rows[b, r, pl.ds(q * 16, 16)] = (
                    rows[b, r, pl.ds(q * 16, 16)] * splat)

    for b in range(NBUF):
        istart(b, b)
    for b in range(NBUF):
        iwait(b)
        decode(b)
        gstart(b)

    @pl.loop(0, NCH - NBUF, step=NBUF)
    def _(c0):
        for b in range(NBUF):
            gwait(b)
            scale(b)
            sstart(b)
        for b in range(NBUF):
            swait(b)
            istart(c0 + NBUF + b, b)
        for b in range(NBUF):
            iwait(b)
            decode(b)
            gstart(b)

    for b in range(NBUF):
        gwait(b)
        scale(b)
        sstart(b)
    for b in range(NBUF):
        swait(b)

    plsc.subcore_barrier()
    pltpu.sync_copy(acc_sh.at[pl.ds(s * NPS, NPS)],
                    out_hbm.at[c, pl.ds(s * NPS, NPS)])


def _make_msg_kernel(wd):
    @functools.partial(
        pl.kernel,
        out_type=jax.ShapeDtypeStruct((NC, NP, wd), _f32),
        mesh=_mesh,
        scratch_types=[
            pltpu.VMEM((NBUF, B), jnp.int32),
            pltpu.VMEM((NBUF, B), jnp.int32),
            pltpu.VMEM((NBUF, B), jnp.int32),
            pltpu.VMEM((NBUF, B), _f32),
            pltpu.VMEM((NBUF, B, wd), _f32),
            pltpu.VMEM((B, wd), _f32),
            pltpu.VMEM_SHARED((NP, wd), _f32),
            pltpu.VMEM_SHARED((NP, wd), _f32),
            pltpu.SemaphoreType.DMA((NBUF,)),
            pltpu.SemaphoreType.DMA((NBUF,)),
            pltpu.SemaphoreType.DMA((NBUF,)),
        ],
        compiler_params=_sc_params,
    )
    def _k(hs_hbm, sd_hbm, w_hbm, out_hbm,
           sd_v, src_v, dst_v, w_v, rows, z_v, acc_sh, hs_sh,
           isem, gsem, ssem):
        _msg_body(wd, hs_hbm, sd_hbm, w_hbm, out_hbm,
                  sd_v, src_v, dst_v, w_v, rows, z_v, acc_sh, hs_sh,
                  isem, gsem, ssem)
    return _k


_msg_kernel_h = _make_msg_kernel(H)
_msg_kernel_c = _make_msg_kernel(CP)


def _tc1_body(x_ref, w1_ref, degp_ref, hs_ref):
    deg = degp_ref[0] + degp_ref[1] + 1.0
    dinv = lax.rsqrt(deg)
    h = jnp.dot(x_ref[...], w1_ref[...], preferred_element_type=_f32)
    hs_ref[...] = h * dinv


def _tc2_body(acc_ref, hs_ref, degp_ref, w2_ref, b1_ref, hs2_ref):
    deg = degp_ref[0] + degp_ref[1] + 1.0
    dinv = lax.rsqrt(deg)
    t = jax.nn.relu(dinv * (acc_ref[0] + acc_ref[1] + hs_ref[...])
                    + b1_ref[...])
    hs2_ref[...] = jnp.dot(t, w2_ref[...], preferred_element_type=_f32) * dinv


def _tc3_body(acc_ref, hs2_ref, degp_ref, b2_ref, lp_ref, xo_ref):
    deg = degp_ref[0] + degp_ref[1] + 1.0
    dinv = lax.rsqrt(deg)
    xo = dinv * (acc_ref[0] + acc_ref[1] + hs2_ref[...]) + b2_ref[...]
    col = lax.broadcasted_iota(jnp.int32, (NP, CP), 1)
    masked = jnp.where(col < C, xo, -1e30)
    m = jnp.max(masked, axis=1, keepdims=True)
    ssum = jnp.sum(jnp.exp(masked - m), axis=1, keepdims=True)
    lp_ref[...] = xo - m - jnp.log(ssum)
    xo_ref[...] = xo


def kernel(x, edge_index, e_w, idx, W1, b1, W2, b2):
    w = jnp.where(idx == 0, jnp.ones((E,), x.dtype), e_w)
    sd = (edge_index[1] << 16) | edge_index[0]
    sd3 = sd.reshape(NW, NCH, B)
    w3 = w.reshape(NW, NCH, B)

    x_pad = jnp.pad(x, ((0, NP - N), (0, 0)))
    w2p = jnp.pad(W2, ((0, 0), (0, CP - C)))
    b1r = b1.reshape(1, H)
    b2r = jnp.pad(b2, (0, CP - C)).reshape(1, CP)

    degp = _deg_kernel(sd3, w3)
    degp3 = degp.reshape(NC, NP, 1)

    hs = pl.pallas_call(
        _tc1_body,
        out_shape=jax.ShapeDtypeStruct((NP, H), _f32),
    )(x_pad, W1, degp3)

    acc1 = _msg_kernel_h(hs, sd3, w3)

    hs2 = pl.pallas_call(
        _tc2_body,
        out_shape=jax.ShapeDtypeStruct((NP, CP), _f32),
    )(acc1, hs, degp3, w2p, b1r)

    acc2 = _msg_kernel_c(hs2, sd3, w3)

    lp, xo = pl.pallas_call(
        _tc3_body,
        out_shape=[jax.ShapeDtypeStruct((NP, CP), _f32),
                   jax.ShapeDtypeStruct((NP, CP), _f32)],
    )(acc2, hs2, degp3, b2r)

    log_probs = lp[:N, :C]
    x_out = xo[:N, :C]
    preg = jnp.asarray(0.0, dtype=_f32)
    return (log_probs, x_out, preg)
